# baseline (speedup 1.0000x reference)
import functools

import jax
import jax.numpy as jnp
from jax import lax
from jax.experimental import pallas as pl
from jax.experimental.pallas import tpu as pltpu
from jax.experimental.pallas import tpu_sc as plsc

NC = 2
NS = 16
NW = NC * NS
CHUNK = 128
NBUF = 4
LOOK = 2


@jax.jit
def _run(x_flat, W):
    N = x_flat.shape[0]
    V, D = W.shape
    n_per_w = N // NW
    n_chunks = n_per_w // CHUNK
    x3 = x_flat.reshape(NW, n_chunks, CHUNK)

    mesh = plsc.VectorSubcoreMesh(core_axis_name="c", subcore_axis_name="s")

    @functools.partial(
        pl.kernel,
        out_type=jax.ShapeDtypeStruct((N, D), jnp.float32),
        mesh=mesh,
        scratch_types=[
            pltpu.VMEM((n_chunks, CHUNK), jnp.int32),
            [pltpu.VMEM((CHUNK, D), jnp.float32) for _ in range(NBUF)],
            [pltpu.SemaphoreType.DMA for _ in range(NBUF)],
            [pltpu.SemaphoreType.DMA for _ in range(NBUF)],
        ],
    )
    def k(x_hbm, w_hbm, out_hbm, idx_v, bufs, gsems, wsems):
        cid = lax.axis_index("c")
        sid = lax.axis_index("s")
        wid = sid * NC + cid
        base = wid * n_per_w

        pltpu.sync_copy(x_hbm.at[wid], idx_v)

        def gather(j, b):
            pltpu.async_copy(w_hbm.at[idx_v.at[j]], bufs[b], gsems[b])

        def wait_gather(b):
            pltpu.make_async_copy(w_hbm.at[idx_v.at[0]], bufs[b],
                                  gsems[b]).wait()

        def write(j, b):
            pltpu.async_copy(bufs[b],
                             out_hbm.at[pl.ds(base + j * CHUNK, CHUNK)],
                             wsems[b])

        def wait_write(b):
            pltpu.make_async_copy(bufs[b], out_hbm.at[pl.ds(base, CHUNK)],
                                  wsems[b]).wait()

        gather(0, 0)
        gather(1, 1)

        for j in (0, 1):
            gather(j + LOOK, (j + LOOK) % NBUF)
            wait_gather(j % NBUF)
            write(j, j % NBUF)

        def step(i, _):
            for b_rel in range(NBUF):
                j = 2 + i * NBUF + b_rel
                bg = (2 + b_rel + LOOK) % NBUF
                wait_write(bg)
                gather_j = j + LOOK
                pltpu.async_copy(w_hbm.at[idx_v.at[gather_j]], bufs[bg],
                                 gsems[bg])
                b = (2 + b_rel) % NBUF
                wait_gather(b)
                write(j, b)
            return 0

        lax.fori_loop(0, (n_chunks - 4) // NBUF, step, 0)

        for j in (n_chunks - 2, n_chunks - 1):
            wait_gather(j % NBUF)
            write(j, j % NBUF)

        for b in range(NBUF):
            wait_write(b)

    return k(x3, W)


def kernel(x, W):
    x = x.astype(jnp.int32)
    B, S = x.shape
    D = W.shape[1]
    out = _run(x.reshape(B * S), W)
    return out.reshape(B, S, D)
